# baseline (speedup 1.0000x reference)
import jax
import jax.numpy as jnp
from jax import lax
from jax.experimental import pallas as pl
from jax.experimental.pallas import tpu as pltpu
from jax.experimental.pallas import tpu_sc as plsc

N = 10000
F = 128
E = 320000
ALPHA = 0.2

NC = 2
NS = 16
NW = NC * NS
EPW = E // NW
K = 80
NCHUNK = EPW // K
NP = 10240
RPT = NP // NS
SPT = NP // NS


def _tc1_body(x_ref, w_ref, a2_ref, h_ref, al_ref, ar_ref):
    h = jnp.dot(x_ref[...], w_ref[...], preferred_element_type=jnp.float32)
    h_ref[...] = h.astype(jnp.bfloat16)
    alrt = lax.dot_general(
        a2_ref[...], h, (((1,), (1,)), ((), ())),
        preferred_element_type=jnp.float32)
    al_ref[...] = alrt[0]
    ar_ref[...] = alrt[1]


def _tc2_body(p_ref, s_ref, o_ref):
    ps = p_ref[0] + p_ref[1]
    ss = s_ref[0] + s_ref[1]
    d = lax.broadcasted_iota(jnp.int32, (F, F), 0)
    cj = lax.broadcasted_iota(jnp.int32, (F, F), 1)
    g = d // 32
    q = d % 32
    orig = g * 32 + jnp.where(q < 16, 2 * q, 2 * (q - 16) + 1)
    perm = (orig == cj).astype(jnp.float32)
    ps = jnp.dot(ps, perm, preferred_element_type=jnp.float32)
    o_ref[...] = ps / (ss + 1e-16)[:, None]


def _sc_body(h_hbm, row_hbm, col_hbm, al_hbm, ar_hbm, p_out, s_out,
             row_v, col_v, a_v, b_v, w_v, rows_v, fbuf_v, zs_v,
             acc_sh, s_sh, gsem0, gsem1, ssem0, ssem1, isem0, isem1):
    gsem = [gsem0, gsem1]
    ssem = [ssem0, ssem1]
    isem = [isem0, isem1]
    c = lax.axis_index("c")
    sid = lax.axis_index("s")
    wid = sid * NC + c

    zero16 = jnp.zeros((16,), jnp.float32)

    def zrow_body(i, carry):
        for j in range(8):
            fbuf_v[0, i, pl.ds(j * 16, 16)] = zero16
        return carry

    lax.fori_loop(0, K, zrow_body, 0)
    for i in range(SPT // 16):
        zs_v[pl.ds(i * 16, 16)] = zero16
    for i in range(RPT // K):
        pltpu.sync_copy(fbuf_v.at[0], acc_sh.at[pl.ds(sid * RPT + i * K, K)])
    pltpu.sync_copy(zs_v, s_sh.at[pl.ds(sid * SPT, SPT)])
    plsc.subcore_barrier()

    e0 = wid * EPW

    def off_of(m):
        return e0 + jnp.minimum(m, NCHUNK - 1) * K

    def idx_start(p, off):
        pltpu.async_copy(row_hbm.at[pl.ds(off, K)], row_v.at[p], isem[p])
        pltpu.async_copy(col_hbm.at[pl.ds(off, K)], col_v.at[p], isem[p])

    def idx_wait(p, off):
        pltpu.make_async_copy(
            row_hbm.at[pl.ds(off, K)], row_v.at[p], isem[p]).wait()
        pltpu.make_async_copy(
            col_hbm.at[pl.ds(off, K)], col_v.at[p], isem[p]).wait()

    def gather_start(p):
        pltpu.async_copy(h_hbm.at[col_v.at[p]], rows_v.at[p], gsem[p])
        pltpu.async_copy(al_hbm.at[row_v.at[p]], a_v.at[p], gsem[p])
        pltpu.async_copy(ar_hbm.at[col_v.at[p]], b_v.at[p], gsem[p])

    def gather_wait(p):
        pltpu.make_async_copy(h_hbm.at[col_v.at[p]], rows_v.at[p],
                              gsem[p]).wait()
        pltpu.make_async_copy(al_hbm.at[row_v.at[p]], a_v.at[p],
                              gsem[p]).wait()
        pltpu.make_async_copy(ar_hbm.at[col_v.at[p]], b_v.at[p],
                              gsem[p]).wait()

    def weights(p):
        for t in range(K // 16):
            sl = pl.ds(t * 16, 16)
            e = a_v[p, sl] + b_v[p, sl]
            e = jnp.where(e < 0.0, e * ALPHA, e)
            w_v[p, sl] = jnp.exp(e)

    def scale(p):
        for t in range(K // 16):
            w16 = w_v[p, pl.ds(t * 16, 16)]
            for l in range(16):
                ws = w16[l]
                ei = t * 16 + l
                for j in range(4):
                    v16 = rows_v[p, ei, pl.ds(j * 16, 16)]
                    v32 = plsc.bitcast(v16, jnp.bfloat16)
                    a, b = plsc.unpack(
                        v32, format=plsc.PackFormat.INTERLEAVED)
                    fbuf_v[p, ei, pl.ds(j * 32, 16)] = a * ws
                    fbuf_v[p, ei, pl.ds(j * 32 + 16, 16)] = b * ws

    def scatter_start(p):
        pltpu.async_copy(fbuf_v.at[p], acc_sh.at[row_v.at[p]], ssem[p],
                         add=True)
        pltpu.async_copy(w_v.at[p], s_sh.at[row_v.at[p]], ssem[p], add=True)

    def scatter_wait(p):
        pltpu.make_async_copy(
            fbuf_v.at[p], acc_sh.at[row_v.at[p]], ssem[p]).wait()
        pltpu.make_async_copy(
            w_v.at[p], s_sh.at[row_v.at[p]], ssem[p]).wait()

    idx_start(0, off_of(0))
    idx_start(1, off_of(1))
    idx_wait(0, off_of(0))
    gather_start(0)

    def pair_body(j, carry):
        ma = 2 * j
        idx_wait(1, off_of(ma + 1))
        gather_start(1)
        gather_wait(0)
        weights(0)
        scale(0)
        scatter_start(0)
        gather_wait(1)
        weights(1)
        scale(1)
        scatter_wait(0)
        idx_start(0, off_of(ma + 2))
        scatter_start(1)
        scatter_wait(1)
        idx_start(1, off_of(ma + 3))
        idx_wait(0, off_of(ma + 2))
        gather_start(0)
        return carry

    lax.fori_loop(0, (NCHUNK - 1) // 2, pair_body, 0)

    gather_wait(0)
    weights(0)
    scale(0)
    scatter_start(0)
    scatter_wait(0)
    idx_wait(1, off_of(NCHUNK))

    plsc.subcore_barrier()

    for i in range(RPT // K):
        r0 = sid * RPT + i * K
        pltpu.sync_copy(acc_sh.at[pl.ds(r0, K)], fbuf_v.at[0])
        pltpu.sync_copy(fbuf_v.at[0], p_out.at[c, pl.ds(r0, K)])
    pltpu.sync_copy(s_sh.at[pl.ds(sid * SPT, SPT)], zs_v)
    pltpu.sync_copy(zs_v, s_out.at[c, pl.ds(sid * SPT, SPT)])


@jax.jit
def kernel(x, edge, W, a_l, a_r):
    a2 = jnp.concatenate(
        [a_l.reshape(1, F), a_r.reshape(1, F)], axis=0)
    row = edge[0].astype(jnp.int32)
    col = edge[1].astype(jnp.int32)

    B = 512
    grid = (N + B - 1) // B
    h, al, ar = pl.pallas_call(
        _tc1_body,
        grid=(grid,),
        in_specs=[
            pl.BlockSpec((B, F), lambda i: (i, 0)),
            pl.BlockSpec((F, F), lambda i: (0, 0)),
            pl.BlockSpec((2, F), lambda i: (0, 0)),
        ],
        out_specs=[
            pl.BlockSpec((B, F), lambda i: (i, 0)),
            pl.BlockSpec((B,), lambda i: (i,)),
            pl.BlockSpec((B,), lambda i: (i,)),
        ],
        out_shape=[
            jax.ShapeDtypeStruct((N, F), jnp.bfloat16),
            jax.ShapeDtypeStruct((N,), jnp.float32),
            jax.ShapeDtypeStruct((N,), jnp.float32),
        ],
    )(x, W, a2)

    mesh = plsc.VectorSubcoreMesh(core_axis_name="c", subcore_axis_name="s")
    sc = pl.kernel(
        _sc_body,
        out_type=[
            jax.ShapeDtypeStruct((NC, NP, F), jnp.float32),
            jax.ShapeDtypeStruct((NC, NP), jnp.float32),
        ],
        mesh=mesh,
        compiler_params=pltpu.CompilerParams(
            needs_layout_passes=False, use_tc_tiling_on_sc=False),
        scratch_types=[
            pltpu.VMEM((2, K), jnp.int32),
            pltpu.VMEM((2, K), jnp.int32),
            pltpu.VMEM((2, K), jnp.float32),
            pltpu.VMEM((2, K), jnp.float32),
            pltpu.VMEM((2, K), jnp.float32),
            pltpu.VMEM((2, K, F // 2), jnp.int32),
            pltpu.VMEM((2, K, F), jnp.float32),
            pltpu.VMEM((SPT,), jnp.float32),
            pltpu.VMEM_SHARED((NP, F), jnp.float32),
            pltpu.VMEM_SHARED((NP,), jnp.float32),
            pltpu.SemaphoreType.DMA,
            pltpu.SemaphoreType.DMA,
            pltpu.SemaphoreType.DMA,
            pltpu.SemaphoreType.DMA,
            pltpu.SemaphoreType.DMA,
            pltpu.SemaphoreType.DMA,
        ],
    )
    h32 = lax.bitcast_convert_type(
        h.reshape(N, F // 2, 2), jnp.int32)
    p, s = sc(h32, row, col, al, ar)

    out_pad = pl.pallas_call(
        _tc2_body,
        grid=(NP // B,),
        in_specs=[
            pl.BlockSpec((2, B, F), lambda i: (0, i, 0)),
            pl.BlockSpec((2, B), lambda i: (0, i)),
        ],
        out_specs=pl.BlockSpec((B, F), lambda i: (i, 0)),
        out_shape=jax.ShapeDtypeStruct((NP, F), jnp.float32),
    )(p, s)
    return out_pad[:N]


# speedup vs baseline: 1.1984x; 1.1984x over previous
import jax
import jax.numpy as jnp
from jax import lax
from jax.experimental import pallas as pl
from jax.experimental.pallas import tpu as pltpu
from jax.experimental.pallas import tpu_sc as plsc

N = 10000
F = 128
E = 320000
ALPHA = 0.2

NC = 2
NS = 16
NW = NC * NS
EPW = E // NW
K = 80
NCHUNK = EPW // K
NP = 10240
RPT = NP // NS
SPT = NP // NS


def _tc1_body(x_ref, w_ref, a2_ref, h_ref, al_ref, ar_ref):
    h = jnp.dot(x_ref[...], w_ref[...], preferred_element_type=jnp.float32)
    h_ref[...] = h
    alrt = lax.dot_general(
        a2_ref[...], h, (((1,), (1,)), ((), ())),
        preferred_element_type=jnp.float32)
    al_ref[...] = alrt[0]
    ar_ref[...] = alrt[1]


def _tc2_body(p_ref, s_ref, o_ref):
    ps = p_ref[0] + p_ref[1]
    ss = s_ref[0] + s_ref[1]
    o_ref[...] = ps / (ss + 1e-16)[:, None]


def _sc_body(h_hbm, rc_hbm, al_hbm, ar_hbm, p_out, s_out,
             al_v, ar_v, rc_v, w_v, rows_v, zs_v,
             acc_sh, s_sh, gsem0, gsem1, ssem0, ssem1, irsem0, irsem1):
    gsem = [gsem0, gsem1]
    ssem = [ssem0, ssem1]
    irsem = [irsem0, irsem1]
    c = lax.axis_index("c")
    sid = lax.axis_index("s")
    wid = sid * NC + c

    pltpu.sync_copy(al_hbm, al_v)
    pltpu.sync_copy(ar_hbm, ar_v)

    zero16 = jnp.zeros((16,), jnp.float32)

    def zrow_body(i, carry):
        for j in range(8):
            rows_v[0, i, pl.ds(j * 16, 16)] = zero16
        return carry

    lax.fori_loop(0, K, zrow_body, 0)
    for i in range(SPT // 16):
        zs_v[pl.ds(i * 16, 16)] = zero16
    for i in range(RPT // K):
        pltpu.sync_copy(rows_v.at[0], acc_sh.at[pl.ds(sid * RPT + i * K, K)])
    pltpu.sync_copy(zs_v, s_sh.at[pl.ds(sid * SPT, SPT)])
    plsc.subcore_barrier()

    gc0 = wid * NCHUNK

    def ir_start(pb, m):
        pltpu.async_copy(rc_hbm.at[:, pl.ds(gc0 + m, 2)], rc_v.at[pb],
                         irsem[pb])

    def ir_wait(pb, m):
        pltpu.make_async_copy(rc_hbm.at[:, pl.ds(gc0 + m, 2)], rc_v.at[pb],
                              irsem[pb]).wait()

    def weights(pb, cp, p):
        for t in range(K // 16):
            sl = pl.ds(t * 16, 16)
            r16 = rc_v[pb, 0, cp, sl]
            c16 = rc_v[pb, 1, cp, sl]
            a = plsc.load_gather(al_v, [r16])
            b = plsc.load_gather(ar_v, [c16])
            e = a + b
            e = jnp.where(e < 0.0, e * ALPHA, e)
            w_v[p, sl] = jnp.exp(e)

    def scale(p):
        def t_body(t, carry):
            w16 = w_v[p, pl.ds(t * 16, 16)]
            for l in range(16):
                ws = w16[l]
                for j in range(8):
                    sl = pl.ds(j * 16, 16)
                    rows_v[p, t * 16 + l, sl] = \
                        rows_v[p, t * 16 + l, sl] * ws
            return carry

        lax.fori_loop(0, K // 16, t_body, 0)

    def g_start(p, pb, cp):
        pltpu.async_copy(h_hbm.at[rc_v.at[pb, 1, cp]], rows_v.at[p],
                         gsem[p])

    def g_wait(p, pb, cp):
        pltpu.make_async_copy(h_hbm.at[rc_v.at[pb, 1, cp]], rows_v.at[p],
                              gsem[p]).wait()

    def sc_start(p, pb, cp):
        pltpu.async_copy(rows_v.at[p], acc_sh.at[rc_v.at[pb, 0, cp]],
                         ssem[p], add=True)
        pltpu.async_copy(w_v.at[p], s_sh.at[rc_v.at[pb, 0, cp]], ssem[p],
                         add=True)

    def sc_wait(p, pb, cp):
        pltpu.make_async_copy(rows_v.at[p], acc_sh.at[rc_v.at[pb, 0, cp]],
                              ssem[p]).wait()
        pltpu.make_async_copy(w_v.at[p], s_sh.at[rc_v.at[pb, 0, cp]],
                              ssem[p]).wait()

    def process(p, pb, cp):
        g_wait(p, pb, cp)
        weights(pb, cp, p)
        scale(p)
        sc_start(p, pb, cp)

    ir_start(0, 0)
    ir_wait(0, 0)
    g_start(0, 0, 0)
    g_start(1, 0, 1)
    ir_start(1, 2)

    def quad_body(i, carry):
        m0 = 4 * i
        process(0, 0, 0)
        process(1, 0, 1)
        sc_wait(0, 0, 0)
        ir_wait(1, m0 + 2)
        g_start(0, 1, 0)
        sc_wait(1, 0, 1)
        ir_start(0, jnp.minimum(m0 + 4, NCHUNK - 1))
        g_start(1, 1, 1)
        process(0, 1, 0)
        process(1, 1, 1)
        sc_wait(0, 1, 0)
        ir_wait(0, jnp.minimum(m0 + 4, NCHUNK - 1))
        g_start(0, 0, 0)
        sc_wait(1, 1, 1)
        ir_start(1, jnp.minimum(m0 + 6, NCHUNK - 1))
        g_start(1, 0, 1)
        return carry

    lax.fori_loop(0, NCHUNK // 4, quad_body, 0)

    process(0, 0, 0)
    sc_wait(0, 0, 0)
    g_wait(1, 0, 1)
    ir_wait(1, NCHUNK - 1)
    plsc.subcore_barrier()

    for i in range(RPT // K):
        r0 = sid * RPT + i * K
        pltpu.sync_copy(acc_sh.at[pl.ds(r0, K)], rows_v.at[0])
        pltpu.sync_copy(rows_v.at[0], p_out.at[c, pl.ds(r0, K)])
    pltpu.sync_copy(s_sh.at[pl.ds(sid * SPT, SPT)], zs_v)
    pltpu.sync_copy(zs_v, s_out.at[c, pl.ds(sid * SPT, SPT)])


@jax.jit
def kernel(x, edge, W, a_l, a_r):
    a2 = jnp.concatenate(
        [a_l.reshape(1, F), a_r.reshape(1, F)], axis=0)
    rcs = edge.astype(jnp.int32)
    rc3 = jnp.concatenate(
        [rcs, jnp.zeros((2, 4 * K), jnp.int32)], axis=1,
    ).reshape(2, E // K + 4, K)

    B = 512
    grid = (N + B - 1) // B
    h, al, ar = pl.pallas_call(
        _tc1_body,
        grid=(grid,),
        in_specs=[
            pl.BlockSpec((B, F), lambda i: (i, 0)),
            pl.BlockSpec((F, F), lambda i: (0, 0)),
            pl.BlockSpec((2, F), lambda i: (0, 0)),
        ],
        out_specs=[
            pl.BlockSpec((B, F), lambda i: (i, 0)),
            pl.BlockSpec((B,), lambda i: (i,)),
            pl.BlockSpec((B,), lambda i: (i,)),
        ],
        out_shape=[
            jax.ShapeDtypeStruct((N, F), jnp.float32),
            jax.ShapeDtypeStruct((N,), jnp.float32),
            jax.ShapeDtypeStruct((N,), jnp.float32),
        ],
    )(x, W, a2)

    mesh = plsc.VectorSubcoreMesh(core_axis_name="c", subcore_axis_name="s")
    sc = pl.kernel(
        _sc_body,
        out_type=[
            jax.ShapeDtypeStruct((NC, NP, F), jnp.float32),
            jax.ShapeDtypeStruct((NC, NP), jnp.float32),
        ],
        mesh=mesh,
        compiler_params=pltpu.CompilerParams(
            needs_layout_passes=False, use_tc_tiling_on_sc=False),
        scratch_types=[
            pltpu.VMEM((N,), jnp.float32),
            pltpu.VMEM((N,), jnp.float32),
            pltpu.VMEM((2, 2, 2, K), jnp.int32),
            pltpu.VMEM((2, K), jnp.float32),
            pltpu.VMEM((2, K, F), jnp.float32),
            pltpu.VMEM((SPT,), jnp.float32),
            pltpu.VMEM_SHARED((NP, F), jnp.float32),
            pltpu.VMEM_SHARED((NP,), jnp.float32),
            pltpu.SemaphoreType.DMA,
            pltpu.SemaphoreType.DMA,
            pltpu.SemaphoreType.DMA,
            pltpu.SemaphoreType.DMA,
            pltpu.SemaphoreType.DMA,
            pltpu.SemaphoreType.DMA,
        ],
    )
    p, s = sc(h, rc3, al, ar)

    out_pad = pl.pallas_call(
        _tc2_body,
        grid=(NP // B,),
        in_specs=[
            pl.BlockSpec((2, B, F), lambda i: (0, i, 0)),
            pl.BlockSpec((2, B), lambda i: (0, i)),
        ],
        out_specs=pl.BlockSpec((B, F), lambda i: (i, 0)),
        out_shape=jax.ShapeDtypeStruct((NP, F), jnp.float32),
    )(p, s)
    return out_pad[:N]


# speedup vs baseline: 1.2042x; 1.0048x over previous
import jax
import jax.numpy as jnp
from jax import lax
from jax.experimental import pallas as pl
from jax.experimental.pallas import tpu as pltpu
from jax.experimental.pallas import tpu_sc as plsc

N = 10000
F = 128
E = 320000
ALPHA = 0.2

NC = 2
NS = 16
NW = NC * NS
EPW = E // NW
K = 80
NCHUNK = EPW // K
NP = 10240
RPT = NP // NS
SPT = NP // NS


def _tc1_body(x_ref, w_ref, a2_ref, h_ref, al_ref, ar_ref):
    h = jnp.dot(x_ref[...], w_ref[...], preferred_element_type=jnp.float32)
    h_ref[...] = h
    alrt = lax.dot_general(
        a2_ref[...], h, (((1,), (1,)), ((), ())),
        preferred_element_type=jnp.float32)
    al_ref[...] = alrt[0]
    ar_ref[...] = alrt[1]


def _tc2_body(p_ref, s_ref, o_ref):
    ps = p_ref[0] + p_ref[1]
    ss = s_ref[0] + s_ref[1]
    o_ref[...] = ps / (ss + 1e-16)[:, None]


def _sc_body(h_hbm, rc_hbm, al_hbm, ar_hbm, p_out, s_out,
             al_v, ar_v, rc_v, w_v, rows_v, zs_v,
             acc_sh, s_sh, gsem0, gsem1, ssem0, ssem1, irsem0, irsem1):
    gsem = [gsem0, gsem1]
    ssem = [ssem0, ssem1]
    irsem = [irsem0, irsem1]
    c = lax.axis_index("c")
    sid = lax.axis_index("s")
    wid = sid * NC + c

    pltpu.sync_copy(al_hbm, al_v)
    pltpu.sync_copy(ar_hbm, ar_v)

    zero16 = jnp.zeros((16,), jnp.float32)

    def zrow_body(i, carry):
        for j in range(8):
            rows_v[0, i, pl.ds(j * 16, 16)] = zero16
        return carry

    lax.fori_loop(0, K, zrow_body, 0)
    for i in range(SPT // 16):
        zs_v[pl.ds(i * 16, 16)] = zero16
    for i in range(RPT // K):
        pltpu.sync_copy(rows_v.at[0], acc_sh.at[pl.ds(sid * RPT + i * K, K)])
    pltpu.sync_copy(zs_v, s_sh.at[pl.ds(sid * SPT, SPT)])
    plsc.subcore_barrier()

    gc0 = wid * NCHUNK

    def ir_start(pb, m):
        pltpu.async_copy(rc_hbm.at[:, pl.ds(gc0 + m, 2)], rc_v.at[pb],
                         irsem[pb])

    def ir_wait(pb, m):
        pltpu.make_async_copy(rc_hbm.at[:, pl.ds(gc0 + m, 2)], rc_v.at[pb],
                              irsem[pb]).wait()

    def weights(pb, cp, p):
        for t in range(K // 16):
            sl = pl.ds(t * 16, 16)
            r16 = rc_v[pb, 0, cp, sl]
            c16 = rc_v[pb, 1, cp, sl]
            a = plsc.load_gather(al_v, [r16])
            b = plsc.load_gather(ar_v, [c16])
            e = a + b
            e = jnp.where(e < 0.0, e * ALPHA, e)
            w_v[p, sl] = jnp.exp(e)

    def scale(p):
        def t_body(t, carry):
            w16 = w_v[p, pl.ds(t * 16, 16)]
            for l in range(16):
                ws = w16[l]
                for j in range(8):
                    sl = pl.ds(j * 16, 16)
                    rows_v[p, t * 16 + l, sl] = \
                        rows_v[p, t * 16 + l, sl] * ws
            return carry

        lax.fori_loop(0, K // 16, t_body, 0)

    def g_start(p, pb, cp):
        pltpu.async_copy(h_hbm.at[rc_v.at[pb, 1, cp]], rows_v.at[p],
                         gsem[p])

    def g_wait(p, pb, cp):
        pltpu.make_async_copy(h_hbm.at[rc_v.at[pb, 1, cp]], rows_v.at[p],
                              gsem[p]).wait()

    def sc_start(p, pb, cp):
        pltpu.async_copy(rows_v.at[p], acc_sh.at[rc_v.at[pb, 0, cp]],
                         ssem[p], add=True)
        pltpu.async_copy(w_v.at[p], s_sh.at[rc_v.at[pb, 0, cp]], ssem[p],
                         add=True)

    def sc_wait(p, pb, cp):
        pltpu.make_async_copy(rows_v.at[p], acc_sh.at[rc_v.at[pb, 0, cp]],
                              ssem[p]).wait()
        pltpu.make_async_copy(w_v.at[p], s_sh.at[rc_v.at[pb, 0, cp]],
                              ssem[p]).wait()

    def process(p, pb, cp):
        g_wait(p, pb, cp)
        weights(pb, cp, p)
        scale(p)
        sc_start(p, pb, cp)

    ir_start(0, 0)
    ir_wait(0, 0)
    g_start(0, 0, 0)
    g_start(1, 0, 1)
    ir_start(1, 2)

    def quad_body(i, carry):
        m0 = 4 * i
        process(0, 0, 0)
        process(1, 0, 1)
        sc_wait(0, 0, 0)
        ir_wait(1, m0 + 2)
        g_start(0, 1, 0)
        sc_wait(1, 0, 1)
        ir_start(0, jnp.minimum(m0 + 4, NCHUNK - 1))
        g_start(1, 1, 1)
        process(0, 1, 0)
        process(1, 1, 1)
        sc_wait(0, 1, 0)
        ir_wait(0, jnp.minimum(m0 + 4, NCHUNK - 1))
        g_start(0, 0, 0)
        sc_wait(1, 1, 1)
        ir_start(1, jnp.minimum(m0 + 6, NCHUNK - 1))
        g_start(1, 0, 1)
        return carry

    lax.fori_loop(0, NCHUNK // 4, quad_body, 0)

    process(0, 0, 0)
    sc_wait(0, 0, 0)
    g_wait(1, 0, 1)
    ir_wait(1, NCHUNK - 1)
    plsc.subcore_barrier()

    for i in range(RPT // K):
        r0 = sid * RPT + i * K
        pltpu.sync_copy(acc_sh.at[pl.ds(r0, K)], p_out.at[c, pl.ds(r0, K)])
    pltpu.sync_copy(s_sh.at[pl.ds(sid * SPT, SPT)], zs_v)
    pltpu.sync_copy(zs_v, s_out.at[c, pl.ds(sid * SPT, SPT)])


@jax.jit
def kernel(x, edge, W, a_l, a_r):
    a2 = jnp.concatenate(
        [a_l.reshape(1, F), a_r.reshape(1, F)], axis=0)
    rcs = edge.astype(jnp.int32)
    rc3 = jnp.concatenate(
        [rcs, jnp.zeros((2, 4 * K), jnp.int32)], axis=1,
    ).reshape(2, E // K + 4, K)

    B = 512
    grid = (N + B - 1) // B
    h, al, ar = pl.pallas_call(
        _tc1_body,
        grid=(grid,),
        in_specs=[
            pl.BlockSpec((B, F), lambda i: (i, 0)),
            pl.BlockSpec((F, F), lambda i: (0, 0)),
            pl.BlockSpec((2, F), lambda i: (0, 0)),
        ],
        out_specs=[
            pl.BlockSpec((B, F), lambda i: (i, 0)),
            pl.BlockSpec((B,), lambda i: (i,)),
            pl.BlockSpec((B,), lambda i: (i,)),
        ],
        out_shape=[
            jax.ShapeDtypeStruct((N, F), jnp.float32),
            jax.ShapeDtypeStruct((N,), jnp.float32),
            jax.ShapeDtypeStruct((N,), jnp.float32),
        ],
    )(x, W, a2)

    mesh = plsc.VectorSubcoreMesh(core_axis_name="c", subcore_axis_name="s")
    sc = pl.kernel(
        _sc_body,
        out_type=[
            jax.ShapeDtypeStruct((NC, NP, F), jnp.float32),
            jax.ShapeDtypeStruct((NC, NP), jnp.float32),
        ],
        mesh=mesh,
        compiler_params=pltpu.CompilerParams(
            needs_layout_passes=False, use_tc_tiling_on_sc=False),
        scratch_types=[
            pltpu.VMEM((N,), jnp.float32),
            pltpu.VMEM((N,), jnp.float32),
            pltpu.VMEM((2, 2, 2, K), jnp.int32),
            pltpu.VMEM((2, K), jnp.float32),
            pltpu.VMEM((2, K, F), jnp.float32),
            pltpu.VMEM((SPT,), jnp.float32),
            pltpu.VMEM_SHARED((NP, F), jnp.float32),
            pltpu.VMEM_SHARED((NP,), jnp.float32),
            pltpu.SemaphoreType.DMA,
            pltpu.SemaphoreType.DMA,
            pltpu.SemaphoreType.DMA,
            pltpu.SemaphoreType.DMA,
            pltpu.SemaphoreType.DMA,
            pltpu.SemaphoreType.DMA,
        ],
    )
    p, s = sc(h, rc3, al, ar)

    out_pad = pl.pallas_call(
        _tc2_body,
        grid=(NP // B,),
        in_specs=[
            pl.BlockSpec((2, B, F), lambda i: (0, i, 0)),
            pl.BlockSpec((2, B), lambda i: (0, i)),
        ],
        out_specs=pl.BlockSpec((B, F), lambda i: (i, 0)),
        out_shape=jax.ShapeDtypeStruct((NP, F), jnp.float32),
    )(p, s)
    return out_pad[:N]


# speedup vs baseline: 1.2603x; 1.0466x over previous
import jax
import jax.numpy as jnp
from jax import lax
from jax.experimental import pallas as pl
from jax.experimental.pallas import tpu as pltpu
from jax.experimental.pallas import tpu_sc as plsc

N = 10000
F = 128
E = 320000
ALPHA = 0.2

NC = 2
NS = 16
NW = NC * NS
EPW = E // NW
K = 80
NCHUNK = EPW // K
NP = 10240
RPT = NP // NS
SPT = NP // NS


def _tc1_body(x_ref, w_ref, a2_ref, h_ref, al_ref, ar_ref):
    h = jnp.dot(x_ref[...], w_ref[...], preferred_element_type=jnp.float32)
    h_ref[...] = h
    alrt = lax.dot_general(
        a2_ref[...], h, (((1,), (1,)), ((), ())),
        preferred_element_type=jnp.float32)
    al_ref[...] = alrt[0]
    ar_ref[...] = alrt[1]


def _tc2_body(p_ref, s_ref, o_ref):
    ps = p_ref[0] + p_ref[1]
    ss = s_ref[0] + s_ref[1]
    o_ref[...] = ps / (ss + 1e-16)[:, None]


def _sc_body(h_hbm, rc_hbm, al_hbm, ar_hbm, p_out, s_out,
             al_v, ar_v, rc_v, w_v, rows_v, zs_v,
             acc_sh, s_sh, gsem0, gsem1, ssem0, ssem1, irsem0, irsem1):
    gsem = [gsem0, gsem1]
    ssem = [ssem0, ssem1]
    irsem = [irsem0, irsem1]
    c = lax.axis_index("c")
    sid = lax.axis_index("s")
    wid = sid * NC + c

    pltpu.sync_copy(al_hbm, al_v)
    pltpu.sync_copy(ar_hbm, ar_v)

    zero16 = jnp.zeros((16,), jnp.float32)

    def zrow_body(i, carry):
        for j in range(8):
            rows_v[0, i, pl.ds(j * 16, 16)] = zero16
        return carry

    lax.fori_loop(0, K, zrow_body, 0)
    for i in range(SPT // 16):
        zs_v[pl.ds(i * 16, 16)] = zero16
    for i in range(RPT // K):
        pltpu.sync_copy(rows_v.at[0], acc_sh.at[pl.ds(sid * RPT + i * K, K)])
    pltpu.sync_copy(zs_v, s_sh.at[pl.ds(sid * SPT, SPT)])
    plsc.subcore_barrier()

    gc0 = wid * NCHUNK

    def ir_start(pb, m):
        pltpu.async_copy(rc_hbm.at[:, pl.ds(gc0 + m, 2)], rc_v.at[pb],
                         irsem[pb])

    def ir_wait(pb, m):
        pltpu.make_async_copy(rc_hbm.at[:, pl.ds(gc0 + m, 2)], rc_v.at[pb],
                              irsem[pb]).wait()

    def weights(pb, cp, p):
        for t in range(K // 16):
            sl = pl.ds(t * 16, 16)
            r16 = rc_v[pb, 0, cp, sl]
            c16 = rc_v[pb, 1, cp, sl]
            a = plsc.load_gather(al_v, [r16])
            b = plsc.load_gather(ar_v, [c16])
            e = a + b
            e = jnp.where(e < 0.0, e * ALPHA, e)
            w_v[p, sl] = jnp.exp(e)

    def scale(p):
        def t_body(t, carry):
            w16 = w_v[p, pl.ds(t * 16, 16)]
            for l in range(16):
                ws = w16[l]
                for j in range(8):
                    sl = pl.ds(j * 16, 16)
                    rows_v[p, t * 16 + l, sl] = \
                        rows_v[p, t * 16 + l, sl] * ws
            return carry

        lax.fori_loop(0, K // 16, t_body, 0)

    def g_start(p, pb, cp):
        pltpu.async_copy(h_hbm.at[rc_v.at[pb, 1, cp]], rows_v.at[p],
                         gsem[p])

    def g_wait(p, pb, cp):
        pltpu.make_async_copy(h_hbm.at[rc_v.at[pb, 1, cp]], rows_v.at[p],
                              gsem[p]).wait()

    def sc_start(p, pb, cp):
        pltpu.async_copy(rows_v.at[p], acc_sh.at[rc_v.at[pb, 0, cp]],
                         ssem[p], add=True)
        pltpu.async_copy(w_v.at[p], s_sh.at[rc_v.at[pb, 0, cp]], ssem[p],
                         add=True)

    def sc_wait(p, pb, cp):
        pltpu.make_async_copy(rows_v.at[p], acc_sh.at[rc_v.at[pb, 0, cp]],
                              ssem[p]).wait()
        pltpu.make_async_copy(w_v.at[p], s_sh.at[rc_v.at[pb, 0, cp]],
                              ssem[p]).wait()

    def process(p, pb, cp):
        weights(pb, cp, p)
        g_wait(p, pb, cp)
        scale(p)
        sc_start(p, pb, cp)

    ir_start(0, 0)
    ir_wait(0, 0)
    g_start(0, 0, 0)
    g_start(1, 0, 1)
    ir_start(1, 2)

    def quad_body(i, carry):
        m0 = 4 * i
        process(0, 0, 0)
        process(1, 0, 1)
        sc_wait(0, 0, 0)
        ir_wait(1, m0 + 2)
        g_start(0, 1, 0)
        sc_wait(1, 0, 1)
        ir_start(0, jnp.minimum(m0 + 4, NCHUNK - 2))
        g_start(1, 1, 1)
        process(0, 1, 0)
        process(1, 1, 1)
        sc_wait(0, 1, 0)
        ir_wait(0, jnp.minimum(m0 + 4, NCHUNK - 2))
        g_start(0, 0, 0)
        sc_wait(1, 1, 1)
        ir_start(1, jnp.minimum(m0 + 6, NCHUNK - 2))
        g_start(1, 0, 1)
        return carry

    lax.fori_loop(0, NCHUNK // 4, quad_body, 0)

    g_wait(0, 0, 0)
    ir_wait(1, NCHUNK - 2)
    process(1, 0, 1)
    sc_wait(1, 0, 1)
    plsc.subcore_barrier()

    for i in range(RPT // K):
        r0 = sid * RPT + i * K
        pltpu.sync_copy(acc_sh.at[pl.ds(r0, K)], p_out.at[c, pl.ds(r0, K)])
    pltpu.sync_copy(s_sh.at[pl.ds(sid * SPT, SPT)], zs_v)
    pltpu.sync_copy(zs_v, s_out.at[c, pl.ds(sid * SPT, SPT)])


@jax.jit
def kernel(x, edge, W, a_l, a_r):
    a2 = jnp.concatenate(
        [a_l.reshape(1, F), a_r.reshape(1, F)], axis=0)
    rc3 = edge.astype(jnp.int32).reshape(2, E // K, K)

    B = 512
    grid = (N + B - 1) // B
    h, al, ar = pl.pallas_call(
        _tc1_body,
        grid=(grid,),
        in_specs=[
            pl.BlockSpec((B, F), lambda i: (i, 0)),
            pl.BlockSpec((F, F), lambda i: (0, 0)),
            pl.BlockSpec((2, F), lambda i: (0, 0)),
        ],
        out_specs=[
            pl.BlockSpec((B, F), lambda i: (i, 0)),
            pl.BlockSpec((B,), lambda i: (i,)),
            pl.BlockSpec((B,), lambda i: (i,)),
        ],
        out_shape=[
            jax.ShapeDtypeStruct((N, F), jnp.float32),
            jax.ShapeDtypeStruct((N,), jnp.float32),
            jax.ShapeDtypeStruct((N,), jnp.float32),
        ],
    )(x, W, a2)

    mesh = plsc.VectorSubcoreMesh(core_axis_name="c", subcore_axis_name="s")
    sc = pl.kernel(
        _sc_body,
        out_type=[
            jax.ShapeDtypeStruct((NC, NP, F), jnp.float32),
            jax.ShapeDtypeStruct((NC, NP), jnp.float32),
        ],
        mesh=mesh,
        compiler_params=pltpu.CompilerParams(
            needs_layout_passes=False, use_tc_tiling_on_sc=False),
        scratch_types=[
            pltpu.VMEM((N,), jnp.float32),
            pltpu.VMEM((N,), jnp.float32),
            pltpu.VMEM((2, 2, 2, K), jnp.int32),
            pltpu.VMEM((2, K), jnp.float32),
            pltpu.VMEM((2, K, F), jnp.float32),
            pltpu.VMEM((SPT,), jnp.float32),
            pltpu.VMEM_SHARED((NP, F), jnp.float32),
            pltpu.VMEM_SHARED((NP,), jnp.float32),
            pltpu.SemaphoreType.DMA,
            pltpu.SemaphoreType.DMA,
            pltpu.SemaphoreType.DMA,
            pltpu.SemaphoreType.DMA,
            pltpu.SemaphoreType.DMA,
            pltpu.SemaphoreType.DMA,
        ],
    )
    p, s = sc(h, rc3, al, ar)

    out_pad = pl.pallas_call(
        _tc2_body,
        grid=(NP // B,),
        in_specs=[
            pl.BlockSpec((2, B, F), lambda i: (0, i, 0)),
            pl.BlockSpec((2, B), lambda i: (0, i)),
        ],
        out_specs=pl.BlockSpec((B, F), lambda i: (i, 0)),
        out_shape=jax.ShapeDtypeStruct((N, F), jnp.float32),
    )(p, s)
    return out_pad


# speedup vs baseline: 1.3477x; 1.0693x over previous
import jax
import jax.numpy as jnp
from jax import lax
from jax.experimental import pallas as pl
from jax.experimental.pallas import tpu as pltpu
from jax.experimental.pallas import tpu_sc as plsc

N = 10000
F = 128
E = 320000
ALPHA = 0.2

NC = 2
NS = 16
NW = NC * NS
EPW = E // NW
K = 80
NCHUNK = EPW // K
NP = 10240
RPT = NP // NS
SPT = NP // NS


def _tc1_body(x_ref, w_ref, a2_ref, h_ref, al_ref, ar_ref):
    h = jnp.dot(x_ref[...], w_ref[...], preferred_element_type=jnp.float32)
    h_ref[...] = h
    alrt = lax.dot_general(
        a2_ref[...], h, (((1,), (1,)), ((), ())),
        preferred_element_type=jnp.float32)
    al_ref[...] = alrt[0]
    ar_ref[...] = alrt[1]


def _tc2_body(p_ref, s_ref, o_ref):
    ps = p_ref[0] + p_ref[1]
    ss = s_ref[0] + s_ref[1]
    o_ref[...] = ps / (ss + 1e-16)[:, None]


def _sc_body(h_hbm, rc_hbm, al_hbm, ar_hbm, p_out, s_out,
             al_v, ar_v, rc_v, w_v, rows_v, zs_v,
             acc_sh, s_sh, gsem0, gsem1, ssem0, ssem1, irsem0, irsem1):
    gsem = [gsem0, gsem1]
    ssem = [ssem0, ssem1]
    irsem = [irsem0, irsem1]
    c = lax.axis_index("c")
    sid = lax.axis_index("s")
    wid = sid * NC + c

    pltpu.async_copy(al_hbm, al_v, irsem0)
    pltpu.async_copy(ar_hbm, ar_v, irsem1)

    zero16 = jnp.zeros((16,), jnp.float32)

    def zrow_body(i, carry):
        for j in range(8):
            rows_v[0, i, pl.ds(j * 16, 16)] = zero16
        return carry

    lax.fori_loop(0, K, zrow_body, 0)
    for i in range(SPT // 16):
        zs_v[pl.ds(i * 16, 16)] = zero16
    for i in range(RPT // K):
        pltpu.sync_copy(rows_v.at[0], acc_sh.at[pl.ds(sid * RPT + i * K, K)])
    pltpu.sync_copy(zs_v, s_sh.at[pl.ds(sid * SPT, SPT)])
    pltpu.make_async_copy(al_hbm, al_v, irsem0).wait()
    pltpu.make_async_copy(ar_hbm, ar_v, irsem1).wait()
    plsc.subcore_barrier()

    gc0 = wid * NCHUNK

    def ir_start(pb, m):
        pltpu.async_copy(rc_hbm.at[:, pl.ds(gc0 + m, 2)], rc_v.at[pb],
                         irsem[pb])

    def ir_wait(pb, m):
        pltpu.make_async_copy(rc_hbm.at[:, pl.ds(gc0 + m, 2)], rc_v.at[pb],
                              irsem[pb]).wait()

    def weights(pb, cp, p):
        for t in range(K // 16):
            sl = pl.ds(t * 16, 16)
            r16 = rc_v[pb, 0, cp, sl]
            c16 = rc_v[pb, 1, cp, sl]
            a = plsc.load_gather(al_v, [r16])
            b = plsc.load_gather(ar_v, [c16])
            e = a + b
            e = jnp.where(e < 0.0, e * ALPHA, e)
            w_v[p, sl] = jnp.exp(e)

    def scale(p):
        def t_body(t, carry):
            w16 = w_v[p, pl.ds(t * 16, 16)]
            for l in range(16):
                ws = w16[l]
                for j in range(8):
                    sl = pl.ds(j * 16, 16)
                    rows_v[p, t * 16 + l, sl] = \
                        rows_v[p, t * 16 + l, sl] * ws
            return carry

        lax.fori_loop(0, K // 16, t_body, 0)

    def g_start(p, pb, cp):
        pltpu.async_copy(h_hbm.at[rc_v.at[pb, 1, cp]], rows_v.at[p],
                         gsem[p])

    def g_wait(p, pb, cp):
        pltpu.make_async_copy(h_hbm.at[rc_v.at[pb, 1, cp]], rows_v.at[p],
                              gsem[p]).wait()

    def sc_start(p, pb, cp):
        pltpu.async_copy(rows_v.at[p], acc_sh.at[rc_v.at[pb, 0, cp]],
                         ssem[p], add=True)
        pltpu.async_copy(w_v.at[p], s_sh.at[rc_v.at[pb, 0, cp]], ssem[p],
                         add=True)

    def sc_wait(p, pb, cp):
        pltpu.make_async_copy(rows_v.at[p], acc_sh.at[rc_v.at[pb, 0, cp]],
                              ssem[p]).wait()
        pltpu.make_async_copy(w_v.at[p], s_sh.at[rc_v.at[pb, 0, cp]],
                              ssem[p]).wait()

    def process(p, pb, cp):
        weights(pb, cp, p)
        g_wait(p, pb, cp)
        scale(p)
        sc_start(p, pb, cp)

    ir_start(0, 0)
    ir_wait(0, 0)
    g_start(0, 0, 0)
    g_start(1, 0, 1)
    ir_start(1, 2)

    def quad_body(i, carry):
        m0 = 4 * i
        process(0, 0, 0)
        process(1, 0, 1)
        sc_wait(0, 0, 0)
        ir_wait(1, m0 + 2)
        g_start(0, 1, 0)
        sc_wait(1, 0, 1)
        ir_start(0, jnp.minimum(m0 + 4, NCHUNK - 2))
        g_start(1, 1, 1)
        process(0, 1, 0)
        process(1, 1, 1)
        sc_wait(0, 1, 0)
        ir_wait(0, jnp.minimum(m0 + 4, NCHUNK - 2))
        g_start(0, 0, 0)
        sc_wait(1, 1, 1)
        ir_start(1, jnp.minimum(m0 + 6, NCHUNK - 2))
        g_start(1, 0, 1)
        return carry

    lax.fori_loop(0, NCHUNK // 4, quad_body, 0)

    g_wait(0, 0, 0)
    ir_wait(1, NCHUNK - 2)
    process(1, 0, 1)
    sc_wait(1, 0, 1)
    plsc.subcore_barrier()

    for i in range(RPT // K):
        r0 = sid * RPT + i * K
        pltpu.sync_copy(acc_sh.at[pl.ds(r0, K)], p_out.at[c, pl.ds(r0, K)])
    pltpu.sync_copy(s_sh.at[pl.ds(sid * SPT, SPT)], zs_v)
    pltpu.sync_copy(zs_v, s_out.at[c, pl.ds(sid * SPT, SPT)])


@jax.jit
def kernel(x, edge, W, a_l, a_r):
    a2 = jnp.concatenate(
        [a_l.reshape(1, F), a_r.reshape(1, F)], axis=0)
    rc3 = edge.astype(jnp.int32).reshape(2, E // K, K)

    B = 1024
    grid = (N + B - 1) // B
    h, al, ar = pl.pallas_call(
        _tc1_body,
        grid=(grid,),
        in_specs=[
            pl.BlockSpec((B, F), lambda i: (i, 0)),
            pl.BlockSpec((F, F), lambda i: (0, 0)),
            pl.BlockSpec((2, F), lambda i: (0, 0)),
        ],
        out_specs=[
            pl.BlockSpec((B, F), lambda i: (i, 0)),
            pl.BlockSpec((B,), lambda i: (i,)),
            pl.BlockSpec((B,), lambda i: (i,)),
        ],
        out_shape=[
            jax.ShapeDtypeStruct((N, F), jnp.float32),
            jax.ShapeDtypeStruct((N,), jnp.float32),
            jax.ShapeDtypeStruct((N,), jnp.float32),
        ],
    )(x, W, a2)

    mesh = plsc.VectorSubcoreMesh(core_axis_name="c", subcore_axis_name="s")
    sc = pl.kernel(
        _sc_body,
        out_type=[
            jax.ShapeDtypeStruct((NC, NP, F), jnp.float32),
            jax.ShapeDtypeStruct((NC, NP), jnp.float32),
        ],
        mesh=mesh,
        compiler_params=pltpu.CompilerParams(
            needs_layout_passes=False, use_tc_tiling_on_sc=False),
        scratch_types=[
            pltpu.VMEM((N,), jnp.float32),
            pltpu.VMEM((N,), jnp.float32),
            pltpu.VMEM((2, 2, 2, K), jnp.int32),
            pltpu.VMEM((2, K), jnp.float32),
            pltpu.VMEM((2, K, F), jnp.float32),
            pltpu.VMEM((SPT,), jnp.float32),
            pltpu.VMEM_SHARED((NP, F), jnp.float32),
            pltpu.VMEM_SHARED((NP,), jnp.float32),
            pltpu.SemaphoreType.DMA,
            pltpu.SemaphoreType.DMA,
            pltpu.SemaphoreType.DMA,
            pltpu.SemaphoreType.DMA,
            pltpu.SemaphoreType.DMA,
            pltpu.SemaphoreType.DMA,
        ],
    )
    p, s = sc(h, rc3, al, ar)

    out_pad = pl.pallas_call(
        _tc2_body,
        grid=(NP // B,),
        in_specs=[
            pl.BlockSpec((2, B, F), lambda i: (0, i, 0)),
            pl.BlockSpec((2, B), lambda i: (0, i)),
        ],
        out_specs=pl.BlockSpec((B, F), lambda i: (i, 0)),
        out_shape=jax.ShapeDtypeStruct((N, F), jnp.float32),
    )(p, s)
    return out_pad


# speedup vs baseline: 1.3777x; 1.0223x over previous
import jax
import jax.numpy as jnp
from jax import lax
from jax.experimental import pallas as pl
from jax.experimental.pallas import tpu as pltpu
from jax.experimental.pallas import tpu_sc as plsc

N = 10000
F = 128
E = 320000
ALPHA = 0.2

NC = 2
NS = 16
NW = NC * NS
EPW = E // NW
K = 80
NCHUNK = EPW // K
NP = 10240
RPT = NP // NS
SPT = NP // NS


def _tc1_body(x_ref, w_ref, a2_ref, h_ref, al_ref, ar_ref):
    h = jnp.dot(x_ref[...], w_ref[...], preferred_element_type=jnp.float32)
    h_ref[...] = h
    alrt = lax.dot_general(
        a2_ref[...], h, (((1,), (1,)), ((), ())),
        preferred_element_type=jnp.float32)
    al_ref[...] = alrt[0]
    ar_ref[...] = alrt[1]


def _tc2_body(p_ref, s_ref, o_ref):
    ps = p_ref[0] + p_ref[1]
    ss = s_ref[0] + s_ref[1]
    o_ref[...] = ps / (ss + 1e-16)[:, None]


def _sc_body(h_hbm, rc_hbm, al_hbm, ar_hbm, p_out, s_out,
             al_v, ar_v, rc_v, w_v, rows_v, zs_v,
             acc_sh, s_sh, gsem0, gsem1, ssem0, ssem1, irsem0, irsem1):
    gsem = [gsem0, gsem1]
    ssem = [ssem0, ssem1]
    irsem = [irsem0, irsem1]
    c = lax.axis_index("c")
    sid = lax.axis_index("s")
    wid = sid * NC + c

    pltpu.async_copy(al_hbm, al_v, irsem0)
    pltpu.async_copy(ar_hbm, ar_v, irsem1)

    zero16 = jnp.zeros((16,), jnp.float32)

    def zrow_body(i, carry):
        for j in range(8):
            rows_v[0, i, pl.ds(j * 16, 16)] = zero16
        return carry

    lax.fori_loop(0, K, zrow_body, 0)
    for i in range(SPT // 16):
        zs_v[pl.ds(i * 16, 16)] = zero16
    for i in range(RPT // K):
        pltpu.sync_copy(rows_v.at[0], acc_sh.at[pl.ds(sid * RPT + i * K, K)])
    pltpu.sync_copy(zs_v, s_sh.at[pl.ds(sid * SPT, SPT)])
    pltpu.make_async_copy(al_hbm, al_v, irsem0).wait()
    pltpu.make_async_copy(ar_hbm, ar_v, irsem1).wait()
    plsc.subcore_barrier()

    gc0 = wid * NCHUNK

    def ir_start(pb, m):
        pltpu.async_copy(rc_hbm.at[:, pl.ds(gc0 + m, 2)], rc_v.at[pb],
                         irsem[pb])

    def ir_wait(pb, m):
        pltpu.make_async_copy(rc_hbm.at[:, pl.ds(gc0 + m, 2)], rc_v.at[pb],
                              irsem[pb]).wait()

    def weights(pb, cp, p):
        for t in range(K // 16):
            sl = pl.ds(t * 16, 16)
            r16 = rc_v[pb, 0, cp, sl]
            c16 = rc_v[pb, 1, cp, sl]
            a = plsc.load_gather(al_v, [r16])
            b = plsc.load_gather(ar_v, [c16])
            e = a + b
            e = jnp.where(e < 0.0, e * ALPHA, e)
            w_v[p, sl] = jnp.exp(e)

    def scale(p):
        def t_body(t, carry):
            w16 = w_v[p, pl.ds(t * 16, 16)]
            for l in range(16):
                ws = w16[l]
                for j in range(8):
                    sl = pl.ds(j * 16, 16)
                    rows_v[p, t * 16 + l, sl] = \
                        rows_v[p, t * 16 + l, sl] * ws
            return carry

        lax.fori_loop(0, K // 16, t_body, 0)

    def g_start(p, pb, cp):
        pltpu.async_copy(h_hbm.at[rc_v.at[pb, 1, cp]], rows_v.at[p],
                         gsem[p])

    def g_wait(p, pb, cp):
        pltpu.make_async_copy(h_hbm.at[rc_v.at[pb, 1, cp]], rows_v.at[p],
                              gsem[p]).wait()

    def sc_start(p, pb, cp):
        pltpu.async_copy(rows_v.at[p], acc_sh.at[rc_v.at[pb, 0, cp]],
                         ssem[p], add=True)
        pltpu.async_copy(w_v.at[p], s_sh.at[rc_v.at[pb, 0, cp]], ssem[p],
                         add=True)

    def sc_wait(p, pb, cp):
        pltpu.make_async_copy(rows_v.at[p], acc_sh.at[rc_v.at[pb, 0, cp]],
                              ssem[p]).wait()
        pltpu.make_async_copy(w_v.at[p], s_sh.at[rc_v.at[pb, 0, cp]],
                              ssem[p]).wait()

    def process(p, pb, cp):
        weights(pb, cp, p)
        g_wait(p, pb, cp)
        scale(p)
        sc_start(p, pb, cp)

    ir_start(0, 0)
    ir_wait(0, 0)
    g_start(0, 0, 0)
    g_start(1, 0, 1)
    ir_start(1, 2)

    def quad_body(i, carry):
        m0 = 4 * i
        process(0, 0, 0)
        process(1, 0, 1)
        sc_wait(0, 0, 0)
        ir_wait(1, m0 + 2)
        g_start(0, 1, 0)
        sc_wait(1, 0, 1)
        ir_start(0, jnp.minimum(m0 + 4, NCHUNK - 2))
        g_start(1, 1, 1)
        process(0, 1, 0)
        process(1, 1, 1)
        sc_wait(0, 1, 0)
        ir_wait(0, jnp.minimum(m0 + 4, NCHUNK - 2))
        g_start(0, 0, 0)
        sc_wait(1, 1, 1)
        ir_start(1, jnp.minimum(m0 + 6, NCHUNK - 2))
        g_start(1, 0, 1)
        return carry

    lax.fori_loop(0, NCHUNK // 4, quad_body, 0)

    g_wait(0, 0, 0)
    ir_wait(1, NCHUNK - 2)
    process(1, 0, 1)
    sc_wait(1, 0, 1)
    plsc.subcore_barrier()

    for i in range(RPT // K):
        r0 = sid * RPT + i * K
        pltpu.sync_copy(acc_sh.at[pl.ds(r0, K)], p_out.at[c, pl.ds(r0, K)])
    pltpu.sync_copy(s_sh.at[pl.ds(sid * SPT, SPT)], zs_v)
    pltpu.sync_copy(zs_v, s_out.at[c, pl.ds(sid * SPT, SPT)])


@jax.jit
def kernel(x, edge, W, a_l, a_r):
    a2 = jnp.concatenate(
        [a_l.reshape(1, F), a_r.reshape(1, F)], axis=0)
    rc3 = edge.astype(jnp.int32).reshape(2, E // K, K)

    B = 2048
    grid = (N + B - 1) // B
    h, al, ar = pl.pallas_call(
        _tc1_body,
        grid=(grid,),
        in_specs=[
            pl.BlockSpec((B, F), lambda i: (i, 0)),
            pl.BlockSpec((F, F), lambda i: (0, 0)),
            pl.BlockSpec((2, F), lambda i: (0, 0)),
        ],
        out_specs=[
            pl.BlockSpec((B, F), lambda i: (i, 0)),
            pl.BlockSpec((B,), lambda i: (i,)),
            pl.BlockSpec((B,), lambda i: (i,)),
        ],
        out_shape=[
            jax.ShapeDtypeStruct((N, F), jnp.float32),
            jax.ShapeDtypeStruct((N,), jnp.float32),
            jax.ShapeDtypeStruct((N,), jnp.float32),
        ],
    )(x, W, a2)

    mesh = plsc.VectorSubcoreMesh(core_axis_name="c", subcore_axis_name="s")
    sc = pl.kernel(
        _sc_body,
        out_type=[
            jax.ShapeDtypeStruct((NC, NP, F), jnp.float32),
            jax.ShapeDtypeStruct((NC, NP), jnp.float32),
        ],
        mesh=mesh,
        compiler_params=pltpu.CompilerParams(
            needs_layout_passes=False, use_tc_tiling_on_sc=False),
        scratch_types=[
            pltpu.VMEM((N,), jnp.float32),
            pltpu.VMEM((N,), jnp.float32),
            pltpu.VMEM((2, 2, 2, K), jnp.int32),
            pltpu.VMEM((2, K), jnp.float32),
            pltpu.VMEM((2, K, F), jnp.float32),
            pltpu.VMEM((SPT,), jnp.float32),
            pltpu.VMEM_SHARED((NP, F), jnp.float32),
            pltpu.VMEM_SHARED((NP,), jnp.float32),
            pltpu.SemaphoreType.DMA,
            pltpu.SemaphoreType.DMA,
            pltpu.SemaphoreType.DMA,
            pltpu.SemaphoreType.DMA,
            pltpu.SemaphoreType.DMA,
            pltpu.SemaphoreType.DMA,
        ],
    )
    p, s = sc(h, rc3, al, ar)

    out_pad = pl.pallas_call(
        _tc2_body,
        grid=(NP // B,),
        in_specs=[
            pl.BlockSpec((2, B, F), lambda i: (0, i, 0)),
            pl.BlockSpec((2, B), lambda i: (0, i)),
        ],
        out_specs=pl.BlockSpec((B, F), lambda i: (i, 0)),
        out_shape=jax.ShapeDtypeStruct((N, F), jnp.float32),
    )(p, s)
    return out_pad


# speedup vs baseline: 1.3950x; 1.0125x over previous
import jax
import jax.numpy as jnp
from jax import lax
from jax.experimental import pallas as pl
from jax.experimental.pallas import tpu as pltpu
from jax.experimental.pallas import tpu_sc as plsc

N = 10000
F = 128
E = 320000
ALPHA = 0.2

NC = 2
NS = 16
NW = NC * NS
EPW = E // NW
K = 80
NCHUNK = EPW // K
NP = 10240
RPT = NP // NS
SPT = NP // NS


def _tc1_body(x_ref, w_ref, a2_ref, h_ref, al_ref, ar_ref):
    h = jnp.dot(x_ref[...], w_ref[...], preferred_element_type=jnp.float32)
    h_ref[...] = h
    alrt = lax.dot_general(
        a2_ref[...], h, (((1,), (1,)), ((), ())),
        preferred_element_type=jnp.float32)
    al_ref[...] = alrt[0]
    ar_ref[...] = alrt[1]


def _tc2_body(p_ref, s_ref, o_ref):
    ps = p_ref[0] + p_ref[1]
    ss = s_ref[0] + s_ref[1]
    o_ref[...] = ps / (ss + 1e-16)[:, None]


def _sc_body(h_hbm, rc_hbm, al_hbm, ar_hbm, p_out, s_out,
             al_v, ar_v, rc_v, w_v, rows_v, zs_v,
             acc_sh, s_sh, gsem0, gsem1, ssem0, ssem1, irsem0, irsem1):
    gsem = [gsem0, gsem1]
    ssem = [ssem0, ssem1]
    irsem = [irsem0, irsem1]
    c = lax.axis_index("c")
    sid = lax.axis_index("s")
    wid = sid * NC + c

    pltpu.async_copy(al_hbm, al_v, irsem0)
    pltpu.async_copy(ar_hbm, ar_v, irsem1)

    zero16 = jnp.zeros((16,), jnp.float32)

    def zrow_body(i, carry):
        for j in range(8):
            rows_v[0, i, pl.ds(j * 16, 16)] = zero16
        return carry

    lax.fori_loop(0, K, zrow_body, 0)
    for i in range(SPT // 16):
        zs_v[pl.ds(i * 16, 16)] = zero16
    for i in range(RPT // K):
        pltpu.sync_copy(rows_v.at[0], acc_sh.at[pl.ds(sid * RPT + i * K, K)])
    pltpu.sync_copy(zs_v, s_sh.at[pl.ds(sid * SPT, SPT)])
    pltpu.make_async_copy(al_hbm, al_v, irsem0).wait()
    pltpu.make_async_copy(ar_hbm, ar_v, irsem1).wait()
    plsc.subcore_barrier()

    gc0 = wid * NCHUNK

    def ir_start(pb, m):
        pltpu.async_copy(rc_hbm.at[:, pl.ds(gc0 + m, 2)], rc_v.at[pb],
                         irsem[pb])

    def ir_wait(pb, m):
        pltpu.make_async_copy(rc_hbm.at[:, pl.ds(gc0 + m, 2)], rc_v.at[pb],
                              irsem[pb]).wait()

    def weights(pb, cp, p):
        for t in range(K // 16):
            sl = pl.ds(t * 16, 16)
            r16 = rc_v[pb, 0, cp, sl]
            c16 = rc_v[pb, 1, cp, sl]
            a = plsc.load_gather(al_v, [r16])
            b = plsc.load_gather(ar_v, [c16])
            e = a + b
            e = jnp.where(e < 0.0, e * ALPHA, e)
            w_v[p, sl] = jnp.exp(e)

    def scale(p):
        def t_body(t, carry):
            w16 = w_v[p, pl.ds(t * 16, 16)]
            for l in range(16):
                ws = w16[l]
                for j in range(8):
                    sl = pl.ds(j * 16, 16)
                    rows_v[p, t * 16 + l, sl] = \
                        rows_v[p, t * 16 + l, sl] * ws
            return carry

        lax.fori_loop(0, K // 16, t_body, 0)

    def g_start(p, pb, cp):
        pltpu.async_copy(h_hbm.at[rc_v.at[pb, 1, cp]], rows_v.at[p],
                         gsem[p])

    def g_wait(p, pb, cp):
        pltpu.make_async_copy(h_hbm.at[rc_v.at[pb, 1, cp]], rows_v.at[p],
                              gsem[p]).wait()

    def sc_start(p, pb, cp):
        pltpu.async_copy(rows_v.at[p], acc_sh.at[rc_v.at[pb, 0, cp]],
                         ssem[p], add=True)
        pltpu.async_copy(w_v.at[p], s_sh.at[rc_v.at[pb, 0, cp]], ssem[p],
                         add=True)

    def sc_wait(p, pb, cp):
        pltpu.make_async_copy(rows_v.at[p], acc_sh.at[rc_v.at[pb, 0, cp]],
                              ssem[p]).wait()
        pltpu.make_async_copy(w_v.at[p], s_sh.at[rc_v.at[pb, 0, cp]],
                              ssem[p]).wait()

    def process(p, pb, cp):
        weights(pb, cp, p)
        g_wait(p, pb, cp)
        scale(p)
        sc_start(p, pb, cp)

    ir_start(0, 0)
    ir_wait(0, 0)
    g_start(0, 0, 0)
    g_start(1, 0, 1)
    ir_start(1, 2)

    def quad_body(i, carry):
        m0 = 4 * i
        process(0, 0, 0)
        process(1, 0, 1)
        sc_wait(0, 0, 0)
        ir_wait(1, m0 + 2)
        g_start(0, 1, 0)
        sc_wait(1, 0, 1)
        ir_start(0, jnp.minimum(m0 + 4, NCHUNK - 2))
        g_start(1, 1, 1)
        process(0, 1, 0)
        process(1, 1, 1)
        sc_wait(0, 1, 0)
        ir_wait(0, jnp.minimum(m0 + 4, NCHUNK - 2))
        g_start(0, 0, 0)
        sc_wait(1, 1, 1)
        ir_start(1, jnp.minimum(m0 + 6, NCHUNK - 2))
        g_start(1, 0, 1)
        return carry

    lax.fori_loop(0, NCHUNK // 4, quad_body, 0)

    g_wait(0, 0, 0)
    ir_wait(1, NCHUNK - 2)
    process(1, 0, 1)
    sc_wait(1, 0, 1)
    plsc.subcore_barrier()

    for i in range(RPT // K):
        r0 = sid * RPT + i * K
        pltpu.sync_copy(acc_sh.at[pl.ds(r0, K)], p_out.at[c, pl.ds(r0, K)])
    pltpu.sync_copy(s_sh.at[pl.ds(sid * SPT, SPT)], zs_v)
    pltpu.sync_copy(zs_v, s_out.at[c, pl.ds(sid * SPT, SPT)])


@jax.jit
def kernel(x, edge, W, a_l, a_r):
    a2 = jnp.concatenate(
        [a_l.reshape(1, F), a_r.reshape(1, F)], axis=0)
    rc3 = edge.astype(jnp.int32).reshape(2, E // K, K)

    B = NP
    grid = (N + B - 1) // B
    h, al, ar = pl.pallas_call(
        _tc1_body,
        grid=(grid,),
        in_specs=[
            pl.BlockSpec((B, F), lambda i: (i, 0)),
            pl.BlockSpec((F, F), lambda i: (0, 0)),
            pl.BlockSpec((2, F), lambda i: (0, 0)),
        ],
        out_specs=[
            pl.BlockSpec((B, F), lambda i: (i, 0)),
            pl.BlockSpec((B,), lambda i: (i,)),
            pl.BlockSpec((B,), lambda i: (i,)),
        ],
        out_shape=[
            jax.ShapeDtypeStruct((N, F), jnp.float32),
            jax.ShapeDtypeStruct((N,), jnp.float32),
            jax.ShapeDtypeStruct((N,), jnp.float32),
        ],
    )(x, W, a2)

    mesh = plsc.VectorSubcoreMesh(core_axis_name="c", subcore_axis_name="s")
    sc = pl.kernel(
        _sc_body,
        out_type=[
            jax.ShapeDtypeStruct((NC, NP, F), jnp.float32),
            jax.ShapeDtypeStruct((NC, NP), jnp.float32),
        ],
        mesh=mesh,
        compiler_params=pltpu.CompilerParams(
            needs_layout_passes=False, use_tc_tiling_on_sc=False),
        scratch_types=[
            pltpu.VMEM((N,), jnp.float32),
            pltpu.VMEM((N,), jnp.float32),
            pltpu.VMEM((2, 2, 2, K), jnp.int32),
            pltpu.VMEM((2, K), jnp.float32),
            pltpu.VMEM((2, K, F), jnp.float32),
            pltpu.VMEM((SPT,), jnp.float32),
            pltpu.VMEM_SHARED((NP, F), jnp.float32),
            pltpu.VMEM_SHARED((NP,), jnp.float32),
            pltpu.SemaphoreType.DMA,
            pltpu.SemaphoreType.DMA,
            pltpu.SemaphoreType.DMA,
            pltpu.SemaphoreType.DMA,
            pltpu.SemaphoreType.DMA,
            pltpu.SemaphoreType.DMA,
        ],
    )
    p, s = sc(h, rc3, al, ar)

    out_pad = pl.pallas_call(
        _tc2_body,
        grid=(NP // B,),
        in_specs=[
            pl.BlockSpec((2, B, F), lambda i: (0, i, 0)),
            pl.BlockSpec((2, B), lambda i: (0, i)),
        ],
        out_specs=pl.BlockSpec((B, F), lambda i: (i, 0)),
        out_shape=jax.ShapeDtypeStruct((N, F), jnp.float32),
    )(p, s)
    return out_pad
